# baseline (speedup 1.0000x reference)
import functools

import jax
import jax.numpy as jnp
from jax import lax
from jax.experimental import pallas as pl
from jax.experimental.pallas import tpu as pltpu
from jax.experimental.pallas import tpu_sc as plsc

_B = 16384
_D = 1024
_NT = 1000
_ROWS = 1024
_NB = _B // _ROWS

_HALF = _B // 2
_NBH = _NB // 2

_NW = 32
_BPW = _HALF // _NW
_CHUNK = 128
_NCH = _BPW // _CHUNK

_sc_mesh = plsc.VectorSubcoreMesh(core_axis_name="c", subcore_axis_name="s")


def _sc_gather_body(base_row, tbl_hbm, idx_hbm, out_hbm, idx_v, ac_v, sem):
    wid = lax.axis_index("s") * 2 + lax.axis_index("c")
    row0 = wid * _NCH
    pltpu.sync_copy(idx_hbm.at[pl.ds(base_row + row0, _NCH)], idx_v)
    copies = [
        pltpu.async_copy(tbl_hbm.at[idx_v.at[j]], ac_v.at[j], sem)
        for j in range(_NCH)
    ]
    for c in copies:
        c.wait()
    pltpu.sync_copy(ac_v, out_hbm.at[pl.ds(row0, _NCH)])


def _make_sc_gather(base_row):
    return functools.partial(
        pl.kernel,
        mesh=_sc_mesh,
        out_type=jax.ShapeDtypeStruct((_HALF // _CHUNK, _CHUNK), jnp.float32),
        scratch_types=[
            pltpu.VMEM((_NCH, _CHUNK), jnp.int32),
            pltpu.VMEM((_NCH, _CHUNK), jnp.float32),
            pltpu.SemaphoreType.DMA,
        ],
    )(functools.partial(_sc_gather_body, base_row))


_sc_gather_lo = _make_sc_gather(0)
_sc_gather_hi = _make_sc_gather(_HALF // _CHUNK)


def _dense_lo_kernel(ac_ref, x0_ref, nz_ref, out_ref):
    ac = ac_ref[...]
    sa = jnp.sqrt(ac)
    sb = jnp.sqrt(1.0 - ac)
    out_ref[...] = sa * x0_ref[...] + sb * nz_ref[...]


def _dense_hi_kernel(prev_ref, ac_ref, x0_ref, nz_ref, out_ref):
    del prev_ref
    ac = ac_ref[...]
    sa = jnp.sqrt(ac)
    sb = jnp.sqrt(1.0 - ac)
    out_ref[...] = sa * x0_ref[...] + sb * nz_ref[...]


@jax.jit
def kernel(x_0, timesteps, noise, alphas_cumprod):
    idx = timesteps.reshape(_B // _CHUNK, _CHUNK)
    ac_lo = _sc_gather_lo(alphas_cumprod, idx).reshape(_HALF, 1)
    ac_hi = _sc_gather_hi(alphas_cumprod, idx).reshape(_HALF, 1)
    out_lo = pl.pallas_call(
        _dense_lo_kernel,
        grid=(_NBH,),
        in_specs=[
            pl.BlockSpec((_ROWS, 1), lambda i: (i, 0)),
            pl.BlockSpec((_ROWS, _D), lambda i: (i, 0)),
            pl.BlockSpec((_ROWS, _D), lambda i: (i, 0)),
        ],
        out_specs=pl.BlockSpec((_ROWS, _D), lambda i: (i, 0)),
        out_shape=jax.ShapeDtypeStruct((_B, _D), jnp.float32),
        compiler_params=pltpu.CompilerParams(
            dimension_semantics=("arbitrary",),
        ),
    )(ac_lo, x_0, noise)
    return pl.pallas_call(
        _dense_hi_kernel,
        grid=(_NBH,),
        in_specs=[
            pl.BlockSpec(memory_space=pl.ANY),
            pl.BlockSpec((_ROWS, 1), lambda i: (i, 0)),
            pl.BlockSpec((_ROWS, _D), lambda i: (i + _NBH, 0)),
            pl.BlockSpec((_ROWS, _D), lambda i: (i + _NBH, 0)),
        ],
        out_specs=pl.BlockSpec((_ROWS, _D), lambda i: (i + _NBH, 0)),
        out_shape=jax.ShapeDtypeStruct((_B, _D), jnp.float32),
        input_output_aliases={0: 0},
        compiler_params=pltpu.CompilerParams(
            dimension_semantics=("arbitrary",),
        ),
    )(out_lo, ac_hi, x_0, noise)


# speedup vs baseline: 1.1374x; 1.1374x over previous
import functools

import jax
import jax.numpy as jnp
from jax import lax
from jax.experimental import pallas as pl
from jax.experimental.pallas import tpu as pltpu
from jax.experimental.pallas import tpu_sc as plsc

_B = 16384
_D = 1024
_NT = 1000
_ROWS = 1024
_NB = _B // _ROWS
_SUB = _ROWS // 128

_NW = 32
_BPW = _B // _NW
_CHUNK = 128
_NCH = _BPW // _CHUNK

_sc_mesh = plsc.VectorSubcoreMesh(core_axis_name="c", subcore_axis_name="s")


@functools.partial(
    pl.kernel,
    mesh=_sc_mesh,
    out_type=jax.ShapeDtypeStruct((_B // _CHUNK, _CHUNK), jnp.float32),
    scratch_types=[
        pltpu.VMEM((_NCH, _CHUNK), jnp.int32),
        pltpu.VMEM((_NCH, _CHUNK), jnp.float32),
        pltpu.SemaphoreType.DMA,
    ],
)
def _sc_gather(tbl_hbm, idx_hbm, out_hbm, idx_v, ac_v, sem):
    wid = lax.axis_index("s") * 2 + lax.axis_index("c")
    row0 = wid * _NCH
    pltpu.sync_copy(idx_hbm.at[pl.ds(row0, _NCH)], idx_v)
    copies = [
        pltpu.async_copy(tbl_hbm.at[idx_v.at[j]], ac_v.at[j], sem)
        for j in range(_NCH)
    ]
    for c in copies:
        c.wait()
    pltpu.sync_copy(ac_v, out_hbm.at[pl.ds(row0, _NCH)])


def _dense_kernel(ac_ref, x0_ref, nz_ref, out_ref):
    act = ac_ref[...]
    rg = lax.broadcasted_iota(jnp.int32, (_ROWS, 128), 0) // 128
    t1 = jnp.zeros((_ROWS, 128), jnp.float32)
    for s in range(_SUB):
        t1 = t1 + jnp.where(rg == s, act[s : s + 1, :], 0.0)
    li = lax.broadcasted_iota(jnp.int32, (_ROWS, 128), 1)
    rm = lax.broadcasted_iota(jnp.int32, (_ROWS, 128), 0) % 128
    ac = jnp.sum(jnp.where(li == rm, t1, 0.0), axis=1, keepdims=True)
    sa = jnp.sqrt(ac)
    sb = jnp.sqrt(1.0 - ac)
    out_ref[...] = sa * x0_ref[...] + sb * nz_ref[...]


@jax.jit
def kernel(x_0, timesteps, noise, alphas_cumprod):
    idx = timesteps.reshape(_B // _CHUNK, _CHUNK)
    ac = _sc_gather(alphas_cumprod, idx)
    return pl.pallas_call(
        _dense_kernel,
        grid=(_NB,),
        in_specs=[
            pl.BlockSpec((_SUB, _CHUNK), lambda i: (i, 0)),
            pl.BlockSpec((_ROWS, _D), lambda i: (i, 0)),
            pl.BlockSpec((_ROWS, _D), lambda i: (i, 0)),
        ],
        out_specs=pl.BlockSpec((_ROWS, _D), lambda i: (i, 0)),
        out_shape=jax.ShapeDtypeStruct((_B, _D), jnp.float32),
        compiler_params=pltpu.CompilerParams(
            dimension_semantics=("arbitrary",),
        ),
    )(ac, x_0, noise)
